# Initial kernel scaffold; baseline (speedup 1.0000x reference)
import functools

import jax
import jax.numpy as jnp
import numpy as np
from jax import lax
from jax.experimental import pallas as pl
from jax.experimental.pallas import tpu as pltpu
from jax.experimental.pallas import tpu_sc as plsc

N = 10000
E = 320000
NFEAT = 128
NHID = 16
HEADS = 8
NOUT = 64
NEG = 0.2

NPAD = 10240
EB = 128
NW = 32
NBATCH = 82
PERW = EB * NBATCH
EPAD = NW * PERW
RPT = NPAD // 16

_f32 = jnp.float32


def _mesh():
    return plsc.VectorSubcoreMesh(core_axis_name="c", subcore_axis_name="s")


def _make_denom_kernel():
    @functools.partial(
        pl.kernel,
        out_type=jax.ShapeDtypeStruct((2, NPAD, 16), _f32),
        mesh=_mesh(),
        scratch_types=[
            pltpu.VMEM((EB,), jnp.int32),
            pltpu.VMEM((EB,), jnp.int32),
            pltpu.VMEM((EB, 16), _f32),
            pltpu.VMEM((EB, 16), _f32),
            pltpu.VMEM((64, 16), _f32),
            pltpu.VMEM_SHARED((NPAD, 16), _f32),
            pltpu.SemaphoreType.DMA,
            pltpu.SemaphoreType.DMA,
        ],
    )
    def denom_k(s_hbm, d_hbm, asrc_hbm, adst_hbm, out_hbm,
                sidx, didx, av, bv, z, dacc, sem1, sem2):
        cid = lax.axis_index("c")
        sid = lax.axis_index("s")

        def zrow(i, _):
            z[i] = jnp.zeros((16,), _f32)
            return 0
        lax.fori_loop(0, 64, zrow, 0)

        def zcopy(j, _):
            pltpu.sync_copy(z, dacc.at[pl.ds(sid * RPT + j * 64, 64), :])
            return 0
        lax.fori_loop(0, RPT // 64, zcopy, 0)
        plsc.subcore_barrier()

        base = (sid * 2 + cid) * PERW

        def body(b, _):
            off = base + b * EB
            pltpu.sync_copy(s_hbm.at[pl.ds(off, EB)], sidx)
            pltpu.sync_copy(d_hbm.at[pl.ds(off, EB)], didx)
            ca = pltpu.async_copy(asrc_hbm.at[sidx], av, sem1)
            cb = pltpu.async_copy(adst_hbm.at[didx], bv, sem2)
            ca.wait()
            cb.wait()

            def inner(e, _):
                v = av[e] + bv[e]
                v = jnp.where(v >= 0.0, v, v * NEG)
                bv[e] = jnp.exp(v)
                return 0
            lax.fori_loop(0, EB, inner, 0)
            pltpu.sync_copy(bv, dacc.at[didx], add=True)
            return 0
        lax.fori_loop(0, NBATCH, body, 0)
        plsc.subcore_barrier()
        pltpu.sync_copy(dacc.at[pl.ds(sid * RPT, RPT), :],
                        out_hbm.at[cid, pl.ds(sid * RPT, RPT), :])

    return denom_k


def _make_msg_kernel(D, NH):
    CH = D // NH
    NCH = CH // 16

    @functools.partial(
        pl.kernel,
        out_type=jax.ShapeDtypeStruct((2, NPAD, D), _f32),
        mesh=_mesh(),
        scratch_types=[
            pltpu.VMEM((EB,), jnp.int32),
            pltpu.VMEM((EB,), jnp.int32),
            pltpu.VMEM((EB, 16), _f32),
            pltpu.VMEM((EB, 16), _f32),
            pltpu.VMEM((EB, 16), _f32),
            pltpu.VMEM((1, 16), _f32),
            pltpu.VMEM((EB, D), _f32),
            pltpu.VMEM((64, D), _f32),
            pltpu.VMEM_SHARED((NPAD, D), _f32),
            pltpu.SemaphoreType.DMA,
            pltpu.SemaphoreType.DMA,
            pltpu.SemaphoreType.DMA,
            pltpu.SemaphoreType.DMA,
        ],
    )
    def msg_k(s_hbm, d_hbm, asrc_hbm, adst_hbm, dn_hbm, h_hbm, out_hbm,
              sidx, didx, av, bv, dnv, cbuf, hv, z, macc,
              sem1, sem2, sem3, sem4):
        cid = lax.axis_index("c")
        sid = lax.axis_index("s")

        def zrow(i, _):
            for k in range(D // 16):
                z[i, pl.ds(k * 16, 16)] = jnp.zeros((16,), _f32)
            return 0
        lax.fori_loop(0, 64, zrow, 0)

        def zcopy(j, _):
            pltpu.sync_copy(z, macc.at[pl.ds(sid * RPT + j * 64, 64), :])
            return 0
        lax.fori_loop(0, RPT // 64, zcopy, 0)
        plsc.subcore_barrier()

        base = (sid * 2 + cid) * PERW

        def body(b, _):
            off = base + b * EB
            pltpu.sync_copy(s_hbm.at[pl.ds(off, EB)], sidx)
            pltpu.sync_copy(d_hbm.at[pl.ds(off, EB)], didx)
            ca = pltpu.async_copy(asrc_hbm.at[sidx], av, sem1)
            cb = pltpu.async_copy(adst_hbm.at[didx], bv, sem2)
            cd = pltpu.async_copy(dn_hbm.at[didx], dnv, sem3)
            chh = pltpu.async_copy(h_hbm.at[sidx], hv, sem4)
            ca.wait()
            cb.wait()
            cd.wait()
            chh.wait()

            def inner(e, _):
                v = av[e] + bv[e]
                v = jnp.where(v >= 0.0, v, v * NEG)
                ex = jnp.exp(v)
                cbuf[0] = ex / (dnv[e] + 1e-16)
                for h in range(NH):
                    c = cbuf[0, h]
                    for k in range(NCH):
                        sl = h * CH + k * 16
                        hv[e, pl.ds(sl, 16)] = hv[e, pl.ds(sl, 16)] * c
                return 0
            lax.fori_loop(0, EB, inner, 0)
            pltpu.sync_copy(hv, macc.at[didx], add=True)
            return 0
        lax.fori_loop(0, NBATCH, body, 0)
        plsc.subcore_barrier()
        pltpu.sync_copy(macc.at[pl.ds(sid * RPT, RPT), :],
                        out_hbm.at[cid, pl.ds(sid * RPT, RPT), :])

    return msg_k


_denom_kernel = _make_denom_kernel()
_msg_kernel_l1 = _make_msg_kernel(HEADS * NHID, HEADS)
_msg_kernel_l2 = _make_msg_kernel(NOUT, 1)


_BLK = 1024
_NBLK = NPAD // _BLK


def _tc1(x_p, W1, A1s, A1d):
    def body(x_ref, w_ref, as_ref, ad_ref, h_out, s_out, d_out):
        h = jnp.dot(x_ref[...], w_ref[...], preferred_element_type=_f32)
        h_out[...] = h
        s_out[...] = jnp.dot(h, as_ref[...], preferred_element_type=_f32)
        d_out[...] = jnp.dot(h, ad_ref[...], preferred_element_type=_f32)

    return pl.pallas_call(
        body,
        grid=(_NBLK,),
        in_specs=[
            pl.BlockSpec((_BLK, NFEAT), lambda i: (i, 0)),
            pl.BlockSpec((NFEAT, NFEAT), lambda i: (0, 0)),
            pl.BlockSpec((NFEAT, 16), lambda i: (0, 0)),
            pl.BlockSpec((NFEAT, 16), lambda i: (0, 0)),
        ],
        out_specs=[
            pl.BlockSpec((_BLK, NFEAT), lambda i: (i, 0)),
            pl.BlockSpec((_BLK, 16), lambda i: (i, 0)),
            pl.BlockSpec((_BLK, 16), lambda i: (i, 0)),
        ],
        out_shape=[
            jax.ShapeDtypeStruct((NPAD, NFEAT), _f32),
            jax.ShapeDtypeStruct((NPAD, 16), _f32),
            jax.ShapeDtypeStruct((NPAD, 16), _f32),
        ],
    )(x_p, W1, A1s, A1d)


def _combine_denom(dpart):
    def body(a_ref, o_ref):
        o_ref[...] = a_ref[0] + a_ref[1]

    return pl.pallas_call(
        body,
        out_shape=jax.ShapeDtypeStruct((NPAD, 16), _f32),
    )(dpart)


def _tc2(a0, a1, b1r, W2, A2s, A2d):
    def body(a0_ref, a1_ref, b_ref, w_ref, as_ref, ad_ref,
             h_out, s_out, d_out):
        i = pl.program_id(0)
        rows = lax.broadcasted_iota(jnp.int32, (_BLK, 1), 0) + i * _BLK
        o = jnp.maximum(a0_ref[...] + a1_ref[...] + b_ref[...], 0.0)
        o = jnp.where(rows < N, o, 0.0)
        h2 = jnp.dot(o, w_ref[...], preferred_element_type=_f32)
        h_out[...] = h2
        s_out[...] = jnp.dot(h2, as_ref[...], preferred_element_type=_f32)
        d_out[...] = jnp.dot(h2, ad_ref[...], preferred_element_type=_f32)

    return pl.pallas_call(
        body,
        grid=(_NBLK,),
        in_specs=[
            pl.BlockSpec((_BLK, NFEAT), lambda i: (i, 0)),
            pl.BlockSpec((_BLK, NFEAT), lambda i: (i, 0)),
            pl.BlockSpec((1, NFEAT), lambda i: (0, 0)),
            pl.BlockSpec((NFEAT, NOUT), lambda i: (0, 0)),
            pl.BlockSpec((NOUT, 16), lambda i: (0, 0)),
            pl.BlockSpec((NOUT, 16), lambda i: (0, 0)),
        ],
        out_specs=[
            pl.BlockSpec((_BLK, NOUT), lambda i: (i, 0)),
            pl.BlockSpec((_BLK, 16), lambda i: (i, 0)),
            pl.BlockSpec((_BLK, 16), lambda i: (i, 0)),
        ],
        out_shape=[
            jax.ShapeDtypeStruct((NPAD, NOUT), _f32),
            jax.ShapeDtypeStruct((NPAD, 16), _f32),
            jax.ShapeDtypeStruct((NPAD, 16), _f32),
        ],
    )(a0, a1, b1r, W2, A2s, A2d)


def _tc3(a0, a1, b2r, bn_gr, bn_br, ln_gr, ln_br):
    def body(a0_ref, a1_ref, b_ref, bng_ref, bnb_ref, lng_ref, lnb_ref,
             o_ref, ssum, ssq):
        p = pl.program_id(0)
        i = pl.program_id(1)
        rows = lax.broadcasted_iota(jnp.int32, (_BLK, 1), 0) + i * _BLK
        h = jnp.maximum(a0_ref[...] + a1_ref[...] + b_ref[...], 0.0)
        h = jnp.where(rows < N, h, 0.0)

        @pl.when(jnp.logical_and(p == 0, i == 0))
        def _():
            ssum[...] = jnp.zeros_like(ssum)
            ssq[...] = jnp.zeros_like(ssq)

        @pl.when(p == 0)
        def _():
            ssum[...] += h.sum(axis=0, keepdims=True)
            ssq[...] += (h * h).sum(axis=0, keepdims=True)

        @pl.when(p == 1)
        def _():
            mu = ssum[...] / float(N)
            var = ssq[...] / float(N) - mu * mu
            hb = (h - mu) / jnp.sqrt(var + 1e-5) * bng_ref[...] + bnb_ref[...]
            lmu = hb.mean(axis=-1, keepdims=True)
            lvar = ((hb - lmu) ** 2).mean(axis=-1, keepdims=True)
            o_ref[...] = ((hb - lmu) / jnp.sqrt(lvar + 1e-5)
                          * lng_ref[...] + lnb_ref[...])

    return pl.pallas_call(
        body,
        grid=(2, _NBLK),
        in_specs=[
            pl.BlockSpec((_BLK, NOUT), lambda p, i: (i, 0)),
            pl.BlockSpec((_BLK, NOUT), lambda p, i: (i, 0)),
            pl.BlockSpec((1, NOUT), lambda p, i: (0, 0)),
            pl.BlockSpec((1, NOUT), lambda p, i: (0, 0)),
            pl.BlockSpec((1, NOUT), lambda p, i: (0, 0)),
            pl.BlockSpec((1, NOUT), lambda p, i: (0, 0)),
            pl.BlockSpec((1, NOUT), lambda p, i: (0, 0)),
        ],
        out_specs=pl.BlockSpec((_BLK, NOUT), lambda p, i: (i, 0)),
        out_shape=jax.ShapeDtypeStruct((NPAD, NOUT), _f32),
        scratch_shapes=[
            pltpu.VMEM((1, NOUT), _f32),
            pltpu.VMEM((1, NOUT), _f32),
        ],
    )(a0, a1, b2r, bn_gr, bn_br, ln_gr, ln_br)


_LOOP = np.arange(N, dtype=np.int32)
_PADE = np.full((EPAD - E - N,), N, dtype=np.int32)
_OH1 = np.zeros((NFEAT, 16), dtype=np.float32)
_OH1[np.arange(NFEAT), np.repeat(np.arange(HEADS), NHID)] = 1.0
_OH2 = np.zeros((NOUT, 16), dtype=np.float32)
_OH2[:, 0] = 1.0


def kernel(x, edge_index, W1, a_s1, a_d1, b1, W2, a_s2, a_d2, b2,
           bn_g, bn_b, ln_g, ln_b):
    s_all = jnp.concatenate([edge_index[0], jnp.asarray(_LOOP),
                             jnp.asarray(_PADE)])
    d_all = jnp.concatenate([edge_index[1], jnp.asarray(_LOOP),
                             jnp.asarray(_PADE)])

    x_p = jnp.pad(x, ((0, NPAD - N), (0, 0)))
    A1s = jnp.asarray(_OH1) * a_s1.reshape(NFEAT)[:, None]
    A1d = jnp.asarray(_OH1) * a_d1.reshape(NFEAT)[:, None]
    A2s = jnp.asarray(_OH2) * a_s2.reshape(NOUT)[:, None]
    A2d = jnp.asarray(_OH2) * a_d2.reshape(NOUT)[:, None]

    h1, as1, ad1 = _tc1(x_p, W1, A1s, A1d)
    dn1p = _denom_kernel(s_all, d_all, as1, ad1)
    dn1 = _combine_denom(dn1p)
    acc1 = _msg_kernel_l1(s_all, d_all, as1, ad1, dn1, h1)

    h2, as2, ad2 = _tc2(acc1[0], acc1[1], b1.reshape(1, NFEAT), W2, A2s, A2d)
    dn2p = _denom_kernel(s_all, d_all, as2, ad2)
    dn2 = _combine_denom(dn2p)
    acc2 = _msg_kernel_l2(s_all, d_all, as2, ad2, dn2, h2)

    out = _tc3(acc2[0], acc2[1], b2.reshape(1, NOUT),
               bn_g.reshape(1, NOUT), bn_b.reshape(1, NOUT),
               ln_g.reshape(1, NOUT), ln_b.reshape(1, NOUT))
    return out[:N]


# speedup vs baseline: 45.2948x; 45.2948x over previous
import functools

import jax
import jax.numpy as jnp
import numpy as np
from jax import lax
from jax.experimental import pallas as pl
from jax.experimental.pallas import tpu as pltpu
from jax.experimental.pallas import tpu_sc as plsc

N = 10000
E = 320000
NFEAT = 128
NHID = 16
HEADS = 8
NOUT = 64
NEG = 0.2

NPAD = 10240
EB = 128
NT = 16
NBATCH = 164
PERT = EB * NBATCH
EPAD = NT * PERT
RPT = NPAD // NT
PBITS = 14

_f32 = jnp.float32


def _make_edge_kernel(DH, NH, HSHIFT):
    CH = DH // NH
    NCH = CH // 16

    @functools.partial(
        pl.kernel,
        out_type=[
            pltpu.HBM((2, NPAD, DH), _f32),
            pltpu.HBM((2, NPAD, 16), _f32),
        ],
        mesh=plsc.VectorSubcoreMesh(core_axis_name="c", subcore_axis_name="s"),
        compiler_params=pltpu.CompilerParams(use_tc_tiling_on_sc=False),
        scratch_types=[
            pltpu.VMEM((NBATCH, EB), jnp.int32),
            pltpu.VMEM((EB,), jnp.int32),
            pltpu.VMEM((EB,), jnp.int32),
            pltpu.VMEM((EB,), jnp.int32),
            pltpu.VMEM((EB,), jnp.int32),
            pltpu.VMEM((EB,), jnp.int32),
            pltpu.VMEM((EB,), jnp.int32),
            pltpu.VMEM((EB, 16), _f32),
            pltpu.VMEM((EB, 16), _f32),
            pltpu.VMEM((EB, 16), _f32),
            pltpu.VMEM((EB, 16), _f32),
            pltpu.VMEM((EB, DH), _f32),
            pltpu.VMEM((EB, DH), _f32),
            pltpu.VMEM((EB, 16), _f32),
            pltpu.VMEM((64, DH), _f32),
            pltpu.VMEM_SHARED((NPAD, DH), _f32),
            pltpu.VMEM_SHARED((NPAD, 16), _f32),
            pltpu.SemaphoreType.DMA,
            pltpu.SemaphoreType.DMA,
            pltpu.SemaphoreType.DMA,
            pltpu.SemaphoreType.DMA,
            pltpu.SemaphoreType.DMA,
            pltpu.SemaphoreType.DMA,
        ],
    )
    def edge_k(pk_hbm, asrc_hbm, adst_hbm, h_hbm, acc_hbm, dn_hbm,
               pidx, si0, si1, di0, di1, so0, so1,
               av0, av1, bv0, bv1, hv0, hv1, exbuf, z,
               macc, dacc, sa0, sa1, sb0, sb1, sh0, sh1):
        cid = lax.axis_index("c")
        sid = lax.axis_index("s")
        si = (si0, si1)
        di = (di0, di1)
        so = (so0, so1)
        av = (av0, av1)
        bv = (bv0, bv1)
        hv = (hv0, hv1)
        sa = (sa0, sa1)
        sb = (sb0, sb1)
        sh = (sh0, sh1)

        def zrow(i, _):
            for k in range(DH // 16):
                z[i, pl.ds(k * 16, 16)] = jnp.zeros((16,), _f32)
            return 0
        lax.fori_loop(0, 64, zrow, 0)

        def zcopy(j, _):
            pltpu.sync_copy(z, macc.at[pl.ds(sid * RPT + j * 64, 64), :])
            return 0
        lax.fori_loop(0, RPT // 64, zcopy, 0)

        def zcopy2(j, _):
            pltpu.sync_copy(z.at[:, pl.ds(0, 16)],
                            dacc.at[pl.ds(sid * RPT + j * 64, 64), :])
            return 0
        lax.fori_loop(0, RPT // 64, zcopy2, 0)

        pltpu.sync_copy(pk_hbm.at[sid], pidx)
        plsc.subcore_barrier()

        off = jnp.full((16,), cid * NPAD, jnp.int32)
        mask = jnp.full((16,), (1 << PBITS) - 1, jnp.int32)
        selv = (lax.iota(jnp.int32, 16)
                + jnp.full((16,), cid * HSHIFT, jnp.int32)) & 15

        def start(b, j):
            for k in range(EB // 16):
                sl = pl.ds(k * 16, 16)
                v = pidx[b, sl]
                s = v & mask
                si[j][sl] = s
                di[j][sl] = lax.shift_right_logical(v, PBITS)
                so[j][sl] = s + off
            pltpu.async_copy(asrc_hbm.at[si[j]], av[j], sa[j])
            pltpu.async_copy(adst_hbm.at[di[j]], bv[j], sb[j])
            pltpu.async_copy(h_hbm.at[so[j]], hv[j], sh[j])

        def wait(j):
            pltpu.make_async_copy(asrc_hbm.at[si[j]], av[j], sa[j]).wait()
            pltpu.make_async_copy(adst_hbm.at[di[j]], bv[j], sb[j]).wait()
            pltpu.make_async_copy(h_hbm.at[so[j]], hv[j], sh[j]).wait()

        start(0, 0)

        def outer(g, _):
            for j in range(2):
                b = g * 2 + j

                @pl.when(b + 1 < NBATCH)
                def _():
                    start(b + 1, 1 - j)
                wait(j)
                avj, bvj, hvj = av[j], bv[j], hv[j]

                def inner(e, _):
                    v = avj[e] + bvj[e]
                    v = jnp.where(v >= 0.0, v, v * NEG)
                    ex = jnp.exp(v)
                    exbuf[e] = ex
                    if HSHIFT:
                        ex = lax.gather(
                            ex, selv[:, None],
                            lax.GatherDimensionNumbers(
                                offset_dims=(),
                                collapsed_slice_dims=(0,),
                                start_index_map=(0,)),
                            slice_sizes=(1,),
                            mode=lax.GatherScatterMode.PROMISE_IN_BOUNDS)
                    for h in range(NH):
                        c = ex[h]
                        for k in range(NCH):
                            sl = h * CH + k * 16
                            hvj[e, pl.ds(sl, 16)] = hvj[e, pl.ds(sl, 16)] * c
                    return 0
                lax.fori_loop(0, EB, inner, 0, unroll=4)
                pltpu.sync_copy(exbuf, dacc.at[di[j]], add=True)
                pltpu.sync_copy(hvj, macc.at[di[j]], add=True)
            return 0
        lax.fori_loop(0, NBATCH // 2, outer, 0)
        plsc.subcore_barrier()
        pltpu.sync_copy(macc.at[pl.ds(sid * RPT, RPT), :],
                        acc_hbm.at[cid, pl.ds(sid * RPT, RPT), :])
        pltpu.sync_copy(dacc.at[pl.ds(sid * RPT, RPT), :],
                        dn_hbm.at[cid, pl.ds(sid * RPT, RPT), :])

    return edge_k


_edge_kernel_l1 = _make_edge_kernel(NFEAT // 2, HEADS // 2, HEADS // 2)
_edge_kernel_l2 = _make_edge_kernel(NOUT // 2, 1, 0)


_BLK = 1024
_NBLK = NPAD // _BLK


def _tc1(x_p, W1, A1s, A1d):
    def body(x_ref, w_ref, as_ref, ad_ref, h_out, s_out, d_out):
        h = jnp.dot(x_ref[...], w_ref[...], preferred_element_type=_f32)
        h_out[0] = h[:, :NFEAT // 2]
        h_out[1] = h[:, NFEAT // 2:]
        s_out[...] = jnp.dot(h, as_ref[...], preferred_element_type=_f32)
        d_out[...] = jnp.dot(h, ad_ref[...], preferred_element_type=_f32)

    return pl.pallas_call(
        body,
        grid=(_NBLK,),
        in_specs=[
            pl.BlockSpec((_BLK, NFEAT), lambda i: (i, 0)),
            pl.BlockSpec((NFEAT, NFEAT), lambda i: (0, 0)),
            pl.BlockSpec((NFEAT, 16), lambda i: (0, 0)),
            pl.BlockSpec((NFEAT, 16), lambda i: (0, 0)),
        ],
        out_specs=[
            pl.BlockSpec((2, _BLK, NFEAT // 2), lambda i: (0, i, 0)),
            pl.BlockSpec((_BLK, 16), lambda i: (i, 0)),
            pl.BlockSpec((_BLK, 16), lambda i: (i, 0)),
        ],
        out_shape=[
            jax.ShapeDtypeStruct((2, NPAD, NFEAT // 2), _f32),
            jax.ShapeDtypeStruct((NPAD, 16), _f32),
            jax.ShapeDtypeStruct((NPAD, 16), _f32),
        ],
    )(x_p, W1, A1s, A1d)


def _tc2(acc, dn, oa, b1r, W2, A2s, A2d):
    def body(a_ref, dn_ref, oa_ref, b_ref, w_ref, as_ref, ad_ref,
             h_out, s_out, d_out):
        i = pl.program_id(0)
        rows = lax.broadcasted_iota(jnp.int32, (_BLK, 1), 0) + i * _BLK
        dnexp = jnp.dot(dn_ref[0], oa_ref[...], preferred_element_type=_f32)
        a = jnp.concatenate([a_ref[0], a_ref[1]], axis=1)
        o = a / (dnexp + 1e-16) + b_ref[...]
        o = jnp.maximum(o, 0.0)
        o = jnp.where(rows < N, o, 0.0)
        h2 = jnp.dot(o, w_ref[...], preferred_element_type=_f32)
        h_out[0] = h2[:, :NOUT // 2]
        h_out[1] = h2[:, NOUT // 2:]
        s_out[...] = jnp.dot(h2, as_ref[...], preferred_element_type=_f32)
        d_out[...] = jnp.dot(h2, ad_ref[...], preferred_element_type=_f32)

    return pl.pallas_call(
        body,
        grid=(_NBLK,),
        in_specs=[
            pl.BlockSpec((2, _BLK, NFEAT // 2), lambda i: (0, i, 0)),
            pl.BlockSpec((2, _BLK, 16), lambda i: (0, i, 0)),
            pl.BlockSpec((16, NFEAT), lambda i: (0, 0)),
            pl.BlockSpec((1, NFEAT), lambda i: (0, 0)),
            pl.BlockSpec((NFEAT, NOUT), lambda i: (0, 0)),
            pl.BlockSpec((NOUT, 16), lambda i: (0, 0)),
            pl.BlockSpec((NOUT, 16), lambda i: (0, 0)),
        ],
        out_specs=[
            pl.BlockSpec((2, _BLK, NOUT // 2), lambda i: (0, i, 0)),
            pl.BlockSpec((_BLK, 16), lambda i: (i, 0)),
            pl.BlockSpec((_BLK, 16), lambda i: (i, 0)),
        ],
        out_shape=[
            jax.ShapeDtypeStruct((2, NPAD, NOUT // 2), _f32),
            jax.ShapeDtypeStruct((NPAD, 16), _f32),
            jax.ShapeDtypeStruct((NPAD, 16), _f32),
        ],
    )(acc, dn, oa, b1r, W2, A2s, A2d)


def _tc3(acc, dn, oa2, b2r, bn_gr, bn_br, ln_gr, ln_br):
    def body(a_ref, dn_ref, oa_ref, b_ref, bng_ref, bnb_ref, lng_ref,
             lnb_ref, o_ref, ssum, ssq):
        p = pl.program_id(0)
        i = pl.program_id(1)
        rows = lax.broadcasted_iota(jnp.int32, (_BLK, 1), 0) + i * _BLK
        dnexp = jnp.dot(dn_ref[0], oa_ref[...], preferred_element_type=_f32)
        a = jnp.concatenate([a_ref[0], a_ref[1]], axis=1)
        h = a / (dnexp + 1e-16) + b_ref[...]
        h = jnp.maximum(h, 0.0)
        h = jnp.where(rows < N, h, 0.0)

        @pl.when(jnp.logical_and(p == 0, i == 0))
        def _():
            ssum[...] = jnp.zeros_like(ssum)
            ssq[...] = jnp.zeros_like(ssq)

        @pl.when(p == 0)
        def _():
            ssum[...] += h.sum(axis=0, keepdims=True)
            ssq[...] += (h * h).sum(axis=0, keepdims=True)

        @pl.when(p == 1)
        def _():
            mu = ssum[...] / float(N)
            var = ssq[...] / float(N) - mu * mu
            hb = (h - mu) / jnp.sqrt(var + 1e-5) * bng_ref[...] + bnb_ref[...]
            lmu = hb.mean(axis=-1, keepdims=True)
            lvar = ((hb - lmu) ** 2).mean(axis=-1, keepdims=True)
            o_ref[...] = ((hb - lmu) / jnp.sqrt(lvar + 1e-5)
                          * lng_ref[...] + lnb_ref[...])

    return pl.pallas_call(
        body,
        grid=(2, _NBLK),
        in_specs=[
            pl.BlockSpec((2, _BLK, NOUT // 2), lambda p, i: (0, i, 0)),
            pl.BlockSpec((2, _BLK, 16), lambda p, i: (0, i, 0)),
            pl.BlockSpec((16, NOUT), lambda p, i: (0, 0)),
            pl.BlockSpec((1, NOUT), lambda p, i: (0, 0)),
            pl.BlockSpec((1, NOUT), lambda p, i: (0, 0)),
            pl.BlockSpec((1, NOUT), lambda p, i: (0, 0)),
            pl.BlockSpec((1, NOUT), lambda p, i: (0, 0)),
            pl.BlockSpec((1, NOUT), lambda p, i: (0, 0)),
        ],
        out_specs=pl.BlockSpec((_BLK, NOUT), lambda p, i: (i, 0)),
        out_shape=jax.ShapeDtypeStruct((N, NOUT), _f32),
        scratch_shapes=[
            pltpu.VMEM((1, NOUT), _f32),
            pltpu.VMEM((1, NOUT), _f32),
        ],
    )(acc, dn, oa2, b2r, bn_gr, bn_br, ln_gr, ln_br)


_LOOP = np.arange(N, dtype=np.int32)
_PADE = np.full((EPAD - E - N,), N, dtype=np.int32)
_OH1 = np.zeros((NFEAT, 16), dtype=np.float32)
_OH1[np.arange(NFEAT), np.repeat(np.arange(HEADS), NHID)] = 1.0
_OH2 = np.zeros((NOUT, 16), dtype=np.float32)
_OH2[:, 0] = 1.0
_OA1 = np.zeros((16, NFEAT), dtype=np.float32)
for _ch in range(NFEAT):
    _OA1[_ch // NHID, _ch] = 1.0
_OA2 = np.zeros((16, NOUT), dtype=np.float32)
_OA2[0, :] = 1.0


def kernel(x, edge_index, W1, a_s1, a_d1, b1, W2, a_s2, a_d2, b2,
           bn_g, bn_b, ln_g, ln_b):
    s_all = jnp.concatenate([edge_index[0], jnp.asarray(_LOOP),
                             jnp.asarray(_PADE)])
    d_all = jnp.concatenate([edge_index[1], jnp.asarray(_LOOP),
                             jnp.asarray(_PADE)])
    pk = (s_all | (d_all << PBITS)).reshape(NT, NBATCH, EB)

    x_p = jnp.pad(x, ((0, NPAD - N), (0, 0)))
    A1s = jnp.asarray(_OH1) * a_s1.reshape(NFEAT)[:, None]
    A1d = jnp.asarray(_OH1) * a_d1.reshape(NFEAT)[:, None]
    A2s = jnp.asarray(_OH2) * a_s2.reshape(NOUT)[:, None]
    A2d = jnp.asarray(_OH2) * a_d2.reshape(NOUT)[:, None]

    h1, as1, ad1 = _tc1(x_p, W1, A1s, A1d)
    acc1, dn1 = _edge_kernel_l1(pk, as1, ad1,
                                h1.reshape(2 * NPAD, NFEAT // 2))

    h2, as2, ad2 = _tc2(acc1, dn1, jnp.asarray(_OA1),
                        b1.reshape(1, NFEAT), W2, A2s, A2d)
    acc2, dn2 = _edge_kernel_l2(pk, as2, ad2,
                                h2.reshape(2 * NPAD, NOUT // 2))

    return _tc3(acc2, dn2, jnp.asarray(_OA2), b2.reshape(1, NOUT),
                bn_g.reshape(1, NOUT), bn_b.reshape(1, NOUT),
                ln_g.reshape(1, NOUT), ln_b.reshape(1, NOUT))
